# baseline (speedup 1.0000x reference)
import jax
import jax.numpy as jnp
from jax import lax
from jax.experimental import pallas as pl
from jax.experimental.pallas import tpu as pltpu
from jax.experimental.pallas import tpu_sc as plsc

N = 10000
D = 128
E = 320000
NC, NS, LANES = 2, 16, 16
NW = NC * NS
EPT = E // NW
CH = 40
NIT = EPT // CH
MPAD = 10240
NBUF = 5
RPT = MPAD // NS
DEG_CH = 2000
DEG_NIT = EPT // DEG_CH
_SC_PARAMS = pltpu.CompilerParams(needs_layout_passes=False)


def _gather16(x, i):
    dnums = lax.GatherDimensionNumbers(
        offset_dims=(), collapsed_slice_dims=(0,), start_index_map=(0,))
    return lax.gather(x, i[:, None], dnums, (1,),
                      mode=lax.GatherScatterMode.PROMISE_IN_BOUNDS)


NDEG = 10240
DRPT = NDEG // NS


def _deg_body(ei_hbm, out_hbm, hist, stage_sh, idxb0, idxb1, acc, tin, isem):
    c = lax.axis_index("c")
    s = lax.axis_index("s")
    wid = c * NS + s
    zeros16 = jnp.zeros((LANES,), jnp.float32)
    ones16 = jnp.ones((LANES,), jnp.float32)

    def zbody(i, _):
        hist[i // (NDEG // LANES), pl.ds((i % (NDEG // LANES)) * LANES, LANES)] = zeros16
        return 0
    lax.fori_loop(0, 2 * (NDEG // LANES), zbody, 0)

    def _load(j, b):
        off = (j // DEG_NIT) * E + wid * EPT + (j % DEG_NIT) * DEG_CH
        pltpu.async_copy(ei_hbm.at[pl.ds(off, DEG_CH)],
                         idxb0 if b == 0 else idxb1, isem.at[b])

    def _drain_idx(b):
        pltpu.make_async_copy(ei_hbm.at[pl.ds(0, DEG_CH)],
                              idxb0 if b == 0 else idxb1,
                              isem.at[b]).wait()

    _load(0, 0)
    _load(1, 1)

    def chunk2(r, _):
        for b in range(2):
            j = 2 * r + b
            buf = idxb0 if b == 0 else idxb1
            _drain_idx(b)
            which = jnp.full((LANES,), j // DEG_NIT, jnp.int32)

            def vec(v, _):
                idx = buf[pl.ds(v * LANES, LANES)]
                plsc.addupdate_scatter(hist, [which, idx], ones16)
                return 0
            lax.fori_loop(0, DEG_CH // LANES, vec, 0)
            pl.when(j + 2 < 2 * DEG_NIT)(lambda: _load(j + 2, b))
        return 0
    lax.fori_loop(0, DEG_NIT, chunk2, 0)

    pltpu.sync_copy(hist, stage_sh.at[s])
    plsc.subcore_barrier()
    base = s * DRPT
    pltpu.sync_copy(stage_sh.at[0, :, pl.ds(base, DRPT)], acc)
    for tt in range(1, NS):
        pltpu.sync_copy(stage_sh.at[tt, :, pl.ds(base, DRPT)], tin)

        def addv(i, _):
            r = i // (DRPT // LANES)
            o = (i % (DRPT // LANES)) * LANES
            acc[r, pl.ds(o, LANES)] = (acc[r, pl.ds(o, LANES)]
                                       + tin[r, pl.ds(o, LANES)])
            return 0
        lax.fori_loop(0, 2 * (DRPT // LANES), addv, 0)
    pltpu.sync_copy(acc, out_hbm.at[c, :, pl.ds(base, DRPT)])


def _degrees(ei):
    mesh = plsc.VectorSubcoreMesh(core_axis_name="c", subcore_axis_name="s")
    return pl.kernel(
        _deg_body,
        out_type=jax.ShapeDtypeStruct((NC, 2, NDEG), jnp.float32),
        mesh=mesh,
        compiler_params=_SC_PARAMS,
        scratch_types=[
            pltpu.VMEM((2, NDEG), jnp.float32),
            pltpu.VMEM_SHARED((NS, 2, NDEG), jnp.float32),
            pltpu.VMEM((DEG_CH,), jnp.int32),
            pltpu.VMEM((DEG_CH,), jnp.int32),
            pltpu.VMEM((2, DRPT), jnp.float32),
            pltpu.VMEM((2, DRPT), jnp.float32),
            pltpu.SemaphoreType.DMA((2,)),
        ],
    )(ei)


def _agg_body(hs_hbm, src_hbm, dst_hbm, out_hbm,
              m_sh, sidxb, didxb, rows, zbuf, isem, gsem, ssem):
    c = lax.axis_index("c")
    s = lax.axis_index("s")
    wid = c * NS + s
    zeros16 = jnp.zeros((LANES,), jnp.float32)

    def zb(i, _):
        zbuf[i // (D // LANES), pl.ds((i % (D // LANES)) * LANES, LANES)] = zeros16
        return 0
    lax.fori_loop(0, 64 * (D // LANES), zb, 0)
    for k in range(RPT // 64):
        pltpu.sync_copy(zbuf, m_sh.at[pl.ds(s * RPT + k * 64, 64)])
    plsc.subcore_barrier()

    def _load(i, b):
        pltpu.async_copy(src_hbm.at[wid, i], sidxb.at[b], isem.at[b])
        pltpu.async_copy(dst_hbm.at[wid, i], didxb.at[b], isem.at[b])

    def _gather(b):
        pltpu.async_copy(hs_hbm.at[sidxb.at[b]], rows.at[b], gsem.at[b])

    def _scatter(b):
        pltpu.async_copy(rows.at[b], m_sh.at[didxb.at[b]], ssem.at[b],
                         add=True)

    def _drain_rows(sem, b):
        pltpu.make_async_copy(hs_hbm.at[pl.ds(0, CH)], rows.at[b],
                              sem.at[b]).wait()

    def _drain_idx(b):
        pltpu.make_async_copy(src_hbm.at[wid, 0], sidxb.at[b],
                              isem.at[b]).wait()
        pltpu.make_async_copy(dst_hbm.at[wid, 0], didxb.at[b],
                              isem.at[b]).wait()

    _load(0, 0)
    _load(1, 1)
    _load(2, 2)
    _load(3, 3)
    for bb in range(3):
        _drain_idx(bb)
        _gather(bb)

    def rnd(r, _):
        for b in range(NBUF):
            i = r * NBUF + b
            b3 = (b + 3) % NBUF
            b4 = (b + 4) % NBUF
            _drain_rows(gsem, b)
            _scatter(b)
            def prefetch():
                pl.when(i >= 1)(lambda: _drain_rows(ssem, b4))
                _load(i + 4, b4)
            pl.when(i + 4 < NIT)(prefetch)
            def launch_gather():
                _drain_idx(b3)
                _gather(b3)
            pl.when(i + 3 < NIT)(launch_gather)
        return 0
    lax.fori_loop(0, NIT // NBUF, rnd, 0)

    for b in range(NBUF):
        _drain_rows(ssem, b)

    plsc.subcore_barrier()
    pltpu.sync_copy(m_sh.at[pl.ds(s * RPT, RPT)],
                    out_hbm.at[c, pl.ds(s * RPT, RPT)])


def _aggregate(hs, src3, dst3):
    mesh = plsc.VectorSubcoreMesh(core_axis_name="c", subcore_axis_name="s")
    return pl.kernel(
        _agg_body,
        out_type=jax.ShapeDtypeStruct((NC, MPAD, D), jnp.float32),
        mesh=mesh,
        compiler_params=_SC_PARAMS,
        scratch_types=[
            pltpu.VMEM_SHARED((MPAD, D), jnp.float32),
            pltpu.VMEM((NBUF, CH), jnp.int32),
            pltpu.VMEM((NBUF, CH), jnp.int32),
            pltpu.VMEM((NBUF, CH, D), jnp.float32),
            pltpu.VMEM((64, D), jnp.float32),
            pltpu.SemaphoreType.DMA((NBUF,)),
            pltpu.SemaphoreType.DMA((NBUF,)),
            pltpu.SemaphoreType.DMA((NBUF,)),
        ],
    )(hs, src3, dst3)


def _dense_body(x_ref, wu1_ref, w2_ref, wu2_ref, bu1_ref, bu2_ref,
                hu_ref, uo_ref):
    x = x_ref[...]
    u = jnp.maximum(jnp.dot(x, wu1_ref[...],
                            preferred_element_type=jnp.float32)
                    + bu1_ref[...], 0.0)
    hu_ref[...] = jnp.dot(u, w2_ref[...], preferred_element_type=jnp.float32)
    uo_ref[...] = jnp.maximum(jnp.dot(u, wu2_ref[...],
                                      preferred_element_type=jnp.float32)
                              + bu2_ref[...], 0.0)


def _dense(x, wu1, w2, wu2, bu1, bu2):
    r = 1000
    return pl.pallas_call(
        _dense_body,
        grid=(N // r,),
        in_specs=[
            pl.BlockSpec((r, D), lambda i: (i, 0)),
            pl.BlockSpec((D, D), lambda i: (0, 0)),
            pl.BlockSpec((D, D), lambda i: (0, 0)),
            pl.BlockSpec((D, D), lambda i: (0, 0)),
            pl.BlockSpec((1, D), lambda i: (0, 0)),
            pl.BlockSpec((1, D), lambda i: (0, 0)),
        ],
        out_specs=[
            pl.BlockSpec((r, D), lambda i: (i, 0)),
            pl.BlockSpec((r, D), lambda i: (i, 0)),
        ],
        out_shape=[
            jax.ShapeDtypeStruct((N, D), jnp.float32),
            jax.ShapeDtypeStruct((N, D), jnp.float32),
        ],
    )(x, wu1, w2, wu2, bu1, bu2)


def _scale_body(hu_ref, degt_ref, hs_ref):
    od = degt_ref[:, 0:1]
    onorm = lax.rsqrt(jnp.maximum(od, 1.0))
    hs_ref[...] = hu_ref[...] * onorm


def _scale(hu, degt):
    r = 1000
    return pl.pallas_call(
        _scale_body,
        grid=(N // r,),
        in_specs=[
            pl.BlockSpec((r, D), lambda i: (i, 0)),
            pl.BlockSpec((r, 2), lambda i: (i, 0)),
        ],
        out_specs=pl.BlockSpec((r, D), lambda i: (i, 0)),
        out_shape=jax.ShapeDtypeStruct((N, D), jnp.float32),
    )(hu, degt)


def _final_body(m_ref, degt_ref, b2_ref, out_ref):
    mm = m_ref[0] + m_ref[1]
    ind = degt_ref[:, 1:2]
    innorm = lax.rsqrt(jnp.maximum(ind, 1.0))
    out_ref[...] = jnp.maximum(mm * innorm + b2_ref[...], 0.0)


def _finalize(m, degt, b2):
    r = 1000
    return pl.pallas_call(
        _final_body,
        grid=(N // r,),
        in_specs=[
            pl.BlockSpec((NC, r, D), lambda i: (0, i, 0)),
            pl.BlockSpec((r, 2), lambda i: (i, 0)),
            pl.BlockSpec((1, D), lambda i: (0, 0)),
        ],
        out_specs=pl.BlockSpec((r, D), lambda i: (i, 0)),
        out_shape=jax.ShapeDtypeStruct((N, D), jnp.float32),
    )(m, degt, b2)


def kernel(x_user, edge_index0, edge_index1, W1, b1, W2, b2, Wu1, bu1,
           Wu2, bu2):
    src = edge_index1[0]
    dst = edge_index1[1]

    deg = _degrees(edge_index1.reshape(2 * E))
    degt = (deg[0] + deg[1]).T

    hu, uo = _dense(x_user, Wu1, W2, Wu2, bu1.reshape(1, D),
                    bu2.reshape(1, D))
    hs = _scale(hu, degt)

    m = _aggregate(hs, src.reshape(NW, NIT, CH),
                   dst.reshape(NW, NIT, CH))
    game = _finalize(m, degt, b2.reshape(1, D))

    return (game, uo)


# speedup vs baseline: 1.0208x; 1.0208x over previous
import jax
import jax.numpy as jnp
from jax import lax
from jax.experimental import pallas as pl
from jax.experimental.pallas import tpu as pltpu
from jax.experimental.pallas import tpu_sc as plsc

N = 10000
D = 128
E = 320000
NC, NS, LANES = 2, 16, 16
NW = NC * NS
EPT = E // NW
CH = 40
NIT = EPT // CH
MPAD = 10240
NBUF = 5
RPT = MPAD // NS
DEG_CH = 2000
DEG_NIT = EPT // DEG_CH
_SC_PARAMS = pltpu.CompilerParams(needs_layout_passes=False)


def _gather16(x, i):
    dnums = lax.GatherDimensionNumbers(
        offset_dims=(), collapsed_slice_dims=(0,), start_index_map=(0,))
    return lax.gather(x, i[:, None], dnums, (1,),
                      mode=lax.GatherScatterMode.PROMISE_IN_BOUNDS)


NDEG = 10240
DRPT = NDEG // NS


def _deg_body(ei_hbm, out_hbm, hist, stage_sh, idxb0, idxb1, acc, tin, isem):
    c = lax.axis_index("c")
    s = lax.axis_index("s")
    wid = c * NS + s
    zeros16 = jnp.zeros((LANES,), jnp.float32)
    ones16 = jnp.ones((LANES,), jnp.float32)

    def zbody(i, _):
        hist[i // (NDEG // LANES), pl.ds((i % (NDEG // LANES)) * LANES, LANES)] = zeros16
        return 0
    lax.fori_loop(0, 2 * (NDEG // LANES), zbody, 0)

    def _load(j, b):
        off = (j // DEG_NIT) * E + wid * EPT + (j % DEG_NIT) * DEG_CH
        pltpu.async_copy(ei_hbm.at[pl.ds(off, DEG_CH)],
                         idxb0 if b == 0 else idxb1, isem.at[b])

    def _drain_idx(b):
        pltpu.make_async_copy(ei_hbm.at[pl.ds(0, DEG_CH)],
                              idxb0 if b == 0 else idxb1,
                              isem.at[b]).wait()

    _load(0, 0)
    _load(1, 1)

    def chunk2(r, _):
        for b in range(2):
            j = 2 * r + b
            buf = idxb0 if b == 0 else idxb1
            _drain_idx(b)
            which = jnp.full((LANES,), j // DEG_NIT, jnp.int32)

            def vec(v, _):
                idx = buf[pl.ds(v * LANES, LANES)]
                plsc.addupdate_scatter(hist, [which, idx], ones16)
                return 0
            lax.fori_loop(0, DEG_CH // LANES, vec, 0)
            pl.when(j + 2 < 2 * DEG_NIT)(lambda: _load(j + 2, b))
        return 0
    lax.fori_loop(0, DEG_NIT, chunk2, 0)

    pltpu.sync_copy(hist, stage_sh.at[s])
    plsc.subcore_barrier()
    base = s * DRPT
    pltpu.sync_copy(stage_sh.at[0, :, pl.ds(base, DRPT)], acc)
    for tt in range(1, NS):
        pltpu.sync_copy(stage_sh.at[tt, :, pl.ds(base, DRPT)], tin)

        def addv(i, _):
            r = i // (DRPT // LANES)
            o = (i % (DRPT // LANES)) * LANES
            acc[r, pl.ds(o, LANES)] = (acc[r, pl.ds(o, LANES)]
                                       + tin[r, pl.ds(o, LANES)])
            return 0
        lax.fori_loop(0, 2 * (DRPT // LANES), addv, 0)
    pltpu.sync_copy(acc, out_hbm.at[c, :, pl.ds(base, DRPT)])


def _degrees(ei):
    mesh = plsc.VectorSubcoreMesh(core_axis_name="c", subcore_axis_name="s")
    return pl.kernel(
        _deg_body,
        out_type=jax.ShapeDtypeStruct((NC, 2, NDEG), jnp.float32),
        mesh=mesh,
        compiler_params=_SC_PARAMS,
        scratch_types=[
            pltpu.VMEM((2, NDEG), jnp.float32),
            pltpu.VMEM_SHARED((NS, 2, NDEG), jnp.float32),
            pltpu.VMEM((DEG_CH,), jnp.int32),
            pltpu.VMEM((DEG_CH,), jnp.int32),
            pltpu.VMEM((2, DRPT), jnp.float32),
            pltpu.VMEM((2, DRPT), jnp.float32),
            pltpu.SemaphoreType.DMA((2,)),
        ],
    )(ei)


def _agg_body(hs_hbm, src_hbm, dst_hbm, out_hbm,
              m_sh, sidxb, didxb, rows, zbuf, isem, gsem, ssem):
    c = lax.axis_index("c")
    s = lax.axis_index("s")
    wid = c * NS + s
    zeros16 = jnp.zeros((LANES,), jnp.float32)

    def zb(i, _):
        zbuf[i // (D // LANES), pl.ds((i % (D // LANES)) * LANES, LANES)] = zeros16
        return 0
    lax.fori_loop(0, 64 * (D // LANES), zb, 0)
    for k in range(RPT // 64):
        pltpu.sync_copy(zbuf, m_sh.at[pl.ds(s * RPT + k * 64, 64)])
    plsc.subcore_barrier()

    def _load(i, b):
        pltpu.async_copy(src_hbm.at[wid, i], sidxb.at[b], isem.at[b])
        pltpu.async_copy(dst_hbm.at[wid, i], didxb.at[b], isem.at[b])

    def _gather(b):
        pltpu.async_copy(hs_hbm.at[sidxb.at[b]], rows.at[b], gsem.at[b])

    def _scatter(b):
        pltpu.async_copy(rows.at[b], m_sh.at[didxb.at[b]], ssem.at[b],
                         add=True)

    def _drain_rows(sem, b):
        pltpu.make_async_copy(hs_hbm.at[pl.ds(0, CH)], rows.at[b],
                              sem.at[b]).wait()

    def _drain_idx(b):
        pltpu.make_async_copy(src_hbm.at[wid, 0], sidxb.at[b],
                              isem.at[b]).wait()
        pltpu.make_async_copy(dst_hbm.at[wid, 0], didxb.at[b],
                              isem.at[b]).wait()

    _load(0, 0)
    _load(1, 1)
    _load(2, 2)
    _load(3, 3)
    for bb in range(3):
        _drain_idx(bb)
        _gather(bb)

    def rnd(r, _):
        for b in range(NBUF):
            i = r * NBUF + b
            b3 = (b + 3) % NBUF
            b4 = (b + 4) % NBUF
            _drain_rows(gsem, b)
            _scatter(b)
            def prefetch():
                pl.when(i >= 1)(lambda: _drain_rows(ssem, b4))
                _load(i + 4, b4)
            pl.when(i + 4 < NIT)(prefetch)
            def launch_gather():
                _drain_idx(b3)
                _gather(b3)
            pl.when(i + 3 < NIT)(launch_gather)
        return 0
    lax.fori_loop(0, NIT // NBUF, rnd, 0)

    for b in range(NBUF):
        _drain_rows(ssem, b)

    plsc.subcore_barrier()
    pltpu.sync_copy(m_sh.at[pl.ds(s * RPT, RPT)],
                    out_hbm.at[c, pl.ds(s * RPT, RPT)])


def _aggregate(hs, src3, dst3):
    mesh = plsc.VectorSubcoreMesh(core_axis_name="c", subcore_axis_name="s")
    return pl.kernel(
        _agg_body,
        out_type=jax.ShapeDtypeStruct((NC, MPAD, D), jnp.float32),
        mesh=mesh,
        compiler_params=_SC_PARAMS,
        scratch_types=[
            pltpu.VMEM_SHARED((MPAD, D), jnp.float32),
            pltpu.VMEM((NBUF, CH), jnp.int32),
            pltpu.VMEM((NBUF, CH), jnp.int32),
            pltpu.VMEM((NBUF, CH, D), jnp.float32),
            pltpu.VMEM((64, D), jnp.float32),
            pltpu.SemaphoreType.DMA((NBUF,)),
            pltpu.SemaphoreType.DMA((NBUF,)),
            pltpu.SemaphoreType.DMA((NBUF,)),
        ],
    )(hs, src3, dst3)


def _dense_body(x_ref, wu1_ref, w2_ref, wu2_ref, bu1_ref, bu2_ref, degt_ref,
                hs_ref, uo_ref):
    x = x_ref[...]
    u = jnp.maximum(jnp.dot(x, wu1_ref[...],
                            preferred_element_type=jnp.float32)
                    + bu1_ref[...], 0.0)
    od = degt_ref[:, 0:1]
    onorm = lax.rsqrt(jnp.maximum(od, 1.0))
    hs_ref[...] = jnp.dot(u, w2_ref[...],
                          preferred_element_type=jnp.float32) * onorm
    uo_ref[...] = jnp.maximum(jnp.dot(u, wu2_ref[...],
                                      preferred_element_type=jnp.float32)
                              + bu2_ref[...], 0.0)


def _dense(x, wu1, w2, wu2, bu1, bu2, degt):
    r = 1000
    return pl.pallas_call(
        _dense_body,
        grid=(N // r,),
        in_specs=[
            pl.BlockSpec((r, D), lambda i: (i, 0)),
            pl.BlockSpec((D, D), lambda i: (0, 0)),
            pl.BlockSpec((D, D), lambda i: (0, 0)),
            pl.BlockSpec((D, D), lambda i: (0, 0)),
            pl.BlockSpec((1, D), lambda i: (0, 0)),
            pl.BlockSpec((1, D), lambda i: (0, 0)),
            pl.BlockSpec((r, 2), lambda i: (i, 0)),
        ],
        out_specs=[
            pl.BlockSpec((r, D), lambda i: (i, 0)),
            pl.BlockSpec((r, D), lambda i: (i, 0)),
        ],
        out_shape=[
            jax.ShapeDtypeStruct((N, D), jnp.float32),
            jax.ShapeDtypeStruct((N, D), jnp.float32),
        ],
    )(x, wu1, w2, wu2, bu1, bu2, degt)


def _final_body(m_ref, degt_ref, b2_ref, out_ref):
    mm = m_ref[0] + m_ref[1]
    ind = degt_ref[:, 1:2]
    innorm = lax.rsqrt(jnp.maximum(ind, 1.0))
    out_ref[...] = jnp.maximum(mm * innorm + b2_ref[...], 0.0)


def _finalize(m, degt, b2):
    r = 1000
    return pl.pallas_call(
        _final_body,
        grid=(N // r,),
        in_specs=[
            pl.BlockSpec((NC, r, D), lambda i: (0, i, 0)),
            pl.BlockSpec((r, 2), lambda i: (i, 0)),
            pl.BlockSpec((1, D), lambda i: (0, 0)),
        ],
        out_specs=pl.BlockSpec((r, D), lambda i: (i, 0)),
        out_shape=jax.ShapeDtypeStruct((N, D), jnp.float32),
    )(m, degt, b2)


def kernel(x_user, edge_index0, edge_index1, W1, b1, W2, b2, Wu1, bu1,
           Wu2, bu2):
    src = edge_index1[0]
    dst = edge_index1[1]

    deg = _degrees(edge_index1.reshape(2 * E))
    degt = (deg[0] + deg[1]).T

    hs, uo = _dense(x_user, Wu1, W2, Wu2, bu1.reshape(1, D),
                    bu2.reshape(1, D), degt)

    m = _aggregate(hs, src.reshape(NW, NIT, CH),
                   dst.reshape(NW, NIT, CH))
    game = _finalize(m, degt, b2.reshape(1, D))

    return (game, uo)


# speedup vs baseline: 1.0209x; 1.0001x over previous
import jax
import jax.numpy as jnp
from jax import lax
from jax.experimental import pallas as pl
from jax.experimental.pallas import tpu as pltpu
from jax.experimental.pallas import tpu_sc as plsc

N = 10000
D = 128
E = 320000
NC, NS, LANES = 2, 16, 16
NW = NC * NS
EPT = E // NW
CH = 40
NIT = EPT // CH
MPAD = 10240
NBUF = 5
RPT = MPAD // NS
DEG_CH = 2000
DEG_NIT = EPT // DEG_CH
_SC_PARAMS = pltpu.CompilerParams(needs_layout_passes=False)


NDEG = 10240
DRPT = NDEG // NS


def _deg_body(ei_hbm, out_hbm, hist, stage_sh, idxb0, idxb1, acc, tin, isem):
    c = lax.axis_index("c")
    s = lax.axis_index("s")
    wid = c * NS + s
    zeros16 = jnp.zeros((LANES,), jnp.float32)
    ones16 = jnp.ones((LANES,), jnp.float32)

    def zbody(i, _):
        hist[i // (NDEG // LANES), pl.ds((i % (NDEG // LANES)) * LANES, LANES)] = zeros16
        return 0
    lax.fori_loop(0, 2 * (NDEG // LANES), zbody, 0)

    def _load(j, b):
        off = (j // DEG_NIT) * E + wid * EPT + (j % DEG_NIT) * DEG_CH
        pltpu.async_copy(ei_hbm.at[pl.ds(off, DEG_CH)],
                         idxb0 if b == 0 else idxb1, isem.at[b])

    def _drain_idx(b):
        pltpu.make_async_copy(ei_hbm.at[pl.ds(0, DEG_CH)],
                              idxb0 if b == 0 else idxb1,
                              isem.at[b]).wait()

    _load(0, 0)
    _load(1, 1)

    def chunk2(r, _):
        for b in range(2):
            j = 2 * r + b
            buf = idxb0 if b == 0 else idxb1
            _drain_idx(b)
            which = jnp.full((LANES,), j // DEG_NIT, jnp.int32)

            def vec(v, _):
                idx = buf[pl.ds(v * LANES, LANES)]
                plsc.addupdate_scatter(hist, [which, idx], ones16)
                return 0
            lax.fori_loop(0, DEG_CH // LANES, vec, 0)
            pl.when(j + 2 < 2 * DEG_NIT)(lambda: _load(j + 2, b))
        return 0
    lax.fori_loop(0, DEG_NIT, chunk2, 0)

    pltpu.sync_copy(hist, stage_sh.at[s])
    plsc.subcore_barrier()
    base = s * DRPT
    pltpu.sync_copy(stage_sh.at[0, :, pl.ds(base, DRPT)], acc)
    for tt in range(1, NS):
        pltpu.sync_copy(stage_sh.at[tt, :, pl.ds(base, DRPT)], tin)

        def addv(i, _):
            r = i // (DRPT // LANES)
            o = (i % (DRPT // LANES)) * LANES
            acc[r, pl.ds(o, LANES)] = (acc[r, pl.ds(o, LANES)]
                                       + tin[r, pl.ds(o, LANES)])
            return 0
        lax.fori_loop(0, 2 * (DRPT // LANES), addv, 0)
    pltpu.sync_copy(acc, out_hbm.at[c, :, pl.ds(base, DRPT)])


def _degrees(ei):
    mesh = plsc.VectorSubcoreMesh(core_axis_name="c", subcore_axis_name="s")
    return pl.kernel(
        _deg_body,
        out_type=jax.ShapeDtypeStruct((NC, 2, NDEG), jnp.float32),
        mesh=mesh,
        compiler_params=_SC_PARAMS,
        scratch_types=[
            pltpu.VMEM((2, NDEG), jnp.float32),
            pltpu.VMEM_SHARED((NS, 2, NDEG), jnp.float32),
            pltpu.VMEM((DEG_CH,), jnp.int32),
            pltpu.VMEM((DEG_CH,), jnp.int32),
            pltpu.VMEM((2, DRPT), jnp.float32),
            pltpu.VMEM((2, DRPT), jnp.float32),
            pltpu.SemaphoreType.DMA((2,)),
        ],
    )(ei)


def _agg_body(hs_hbm, src_hbm, dst_hbm, out_hbm,
              m_sh, sidxb, didxb, rows, zbuf, isem, gsem, ssem):
    c = lax.axis_index("c")
    s = lax.axis_index("s")
    wid = c * NS + s
    zeros16 = jnp.zeros((LANES,), jnp.float32)

    def zb(i, _):
        zbuf[i // (D // LANES), pl.ds((i % (D // LANES)) * LANES, LANES)] = zeros16
        return 0
    lax.fori_loop(0, 64 * (D // LANES), zb, 0)
    for k in range(RPT // 64):
        pltpu.sync_copy(zbuf, m_sh.at[pl.ds(s * RPT + k * 64, 64)])
    plsc.subcore_barrier()

    def _load(i, b):
        pltpu.async_copy(src_hbm.at[wid, i], sidxb.at[b], isem.at[b])
        pltpu.async_copy(dst_hbm.at[wid, i], didxb.at[b], isem.at[b])

    def _gather(b):
        pltpu.async_copy(hs_hbm.at[sidxb.at[b]], rows.at[b], gsem.at[b])

    def _scatter(b):
        pltpu.async_copy(rows.at[b], m_sh.at[didxb.at[b]], ssem.at[b],
                         add=True)

    def _drain_rows(sem, b):
        pltpu.make_async_copy(hs_hbm.at[pl.ds(0, CH)], rows.at[b],
                              sem.at[b]).wait()

    def _drain_idx(b):
        pltpu.make_async_copy(src_hbm.at[wid, 0], sidxb.at[b],
                              isem.at[b]).wait()
        pltpu.make_async_copy(dst_hbm.at[wid, 0], didxb.at[b],
                              isem.at[b]).wait()

    _load(0, 0)
    _load(1, 1)
    _load(2, 2)
    _load(3, 3)
    for bb in range(3):
        _drain_idx(bb)
        _gather(bb)

    def rnd(r, _):
        for b in range(NBUF):
            i = r * NBUF + b
            b3 = (b + 3) % NBUF
            b4 = (b + 4) % NBUF
            _drain_rows(gsem, b)
            _scatter(b)
            def prefetch():
                pl.when(i >= 1)(lambda: _drain_rows(ssem, b4))
                _load(i + 4, b4)
            pl.when(i + 4 < NIT)(prefetch)
            def launch_gather():
                _drain_idx(b3)
                _gather(b3)
            pl.when(i + 3 < NIT)(launch_gather)
        return 0
    lax.fori_loop(0, NIT // NBUF, rnd, 0)

    for b in range(NBUF):
        _drain_rows(ssem, b)

    plsc.subcore_barrier()
    pltpu.sync_copy(m_sh.at[pl.ds(s * RPT, RPT)],
                    out_hbm.at[c, pl.ds(s * RPT, RPT)])


def _aggregate(hs, src3, dst3):
    mesh = plsc.VectorSubcoreMesh(core_axis_name="c", subcore_axis_name="s")
    return pl.kernel(
        _agg_body,
        out_type=jax.ShapeDtypeStruct((NC, MPAD, D), jnp.float32),
        mesh=mesh,
        compiler_params=_SC_PARAMS,
        scratch_types=[
            pltpu.VMEM_SHARED((MPAD, D), jnp.float32),
            pltpu.VMEM((NBUF, CH), jnp.int32),
            pltpu.VMEM((NBUF, CH), jnp.int32),
            pltpu.VMEM((NBUF, CH, D), jnp.float32),
            pltpu.VMEM((64, D), jnp.float32),
            pltpu.SemaphoreType.DMA((NBUF,)),
            pltpu.SemaphoreType.DMA((NBUF,)),
            pltpu.SemaphoreType.DMA((NBUF,)),
        ],
    )(hs, src3, dst3)


def _dense_body(x_ref, wu1_ref, w2_ref, wu2_ref, bu1_ref, bu2_ref, degt_ref,
                hs_ref, uo_ref):
    x = x_ref[...]
    u = jnp.maximum(jnp.dot(x, wu1_ref[...],
                            preferred_element_type=jnp.float32)
                    + bu1_ref[...], 0.0)
    od = degt_ref[:, 0:1]
    onorm = lax.rsqrt(jnp.maximum(od, 1.0))
    hs_ref[...] = jnp.dot(u, w2_ref[...],
                          preferred_element_type=jnp.float32) * onorm
    uo_ref[...] = jnp.maximum(jnp.dot(u, wu2_ref[...],
                                      preferred_element_type=jnp.float32)
                              + bu2_ref[...], 0.0)


def _dense(x, wu1, w2, wu2, bu1, bu2, degt):
    r = 1000
    return pl.pallas_call(
        _dense_body,
        grid=(N // r,),
        in_specs=[
            pl.BlockSpec((r, D), lambda i: (i, 0)),
            pl.BlockSpec((D, D), lambda i: (0, 0)),
            pl.BlockSpec((D, D), lambda i: (0, 0)),
            pl.BlockSpec((D, D), lambda i: (0, 0)),
            pl.BlockSpec((1, D), lambda i: (0, 0)),
            pl.BlockSpec((1, D), lambda i: (0, 0)),
            pl.BlockSpec((r, 2), lambda i: (i, 0)),
        ],
        out_specs=[
            pl.BlockSpec((r, D), lambda i: (i, 0)),
            pl.BlockSpec((r, D), lambda i: (i, 0)),
        ],
        out_shape=[
            jax.ShapeDtypeStruct((N, D), jnp.float32),
            jax.ShapeDtypeStruct((N, D), jnp.float32),
        ],
    )(x, wu1, w2, wu2, bu1, bu2, degt)


def _final_body(m_ref, degt_ref, b2_ref, out_ref):
    mm = m_ref[0] + m_ref[1]
    ind = degt_ref[:, 1:2]
    innorm = lax.rsqrt(jnp.maximum(ind, 1.0))
    out_ref[...] = jnp.maximum(mm * innorm + b2_ref[...], 0.0)


def _finalize(m, degt, b2):
    r = 1000
    return pl.pallas_call(
        _final_body,
        grid=(N // r,),
        in_specs=[
            pl.BlockSpec((NC, r, D), lambda i: (0, i, 0)),
            pl.BlockSpec((r, 2), lambda i: (i, 0)),
            pl.BlockSpec((1, D), lambda i: (0, 0)),
        ],
        out_specs=pl.BlockSpec((r, D), lambda i: (i, 0)),
        out_shape=jax.ShapeDtypeStruct((N, D), jnp.float32),
    )(m, degt, b2)


def kernel(x_user, edge_index0, edge_index1, W1, b1, W2, b2, Wu1, bu1,
           Wu2, bu2):
    src = edge_index1[0]
    dst = edge_index1[1]

    deg = _degrees(edge_index1.reshape(2 * E))
    degt = (deg[0] + deg[1]).T

    hs, uo = _dense(x_user, Wu1, W2, Wu2, bu1.reshape(1, D),
                    bu2.reshape(1, D), degt)

    m = _aggregate(hs, src.reshape(NW, NIT, CH),
                   dst.reshape(NW, NIT, CH))
    game = _finalize(m, degt, b2.reshape(1, D))

    return (game, uo)
